# Initial kernel scaffold; baseline (speedup 1.0000x reference)
import jax
import jax.numpy as jnp
from jax.experimental import pallas as pl


def kernel(x, tok_table, pos_table):
    raise NotImplementedError("write your pallas kernel here")


# speedup vs baseline: 1.3939x; 1.3939x over previous
import functools

import jax
import jax.numpy as jnp
from jax import lax
from jax.experimental import pallas as pl
from jax.experimental.pallas import tpu as pltpu
from jax.experimental.pallas import tpu_sc as plsc

EMBED = 32
SEQ = 200
BATCH = 4096
TOTAL = BATCH * SEQ
NC = 2
NS = 16
NW = NC * NS
PER_W = TOTAL // NW
CHUNK = 800
NCHUNK = PER_W // CHUNK
SEQ_PER_CHUNK = CHUNK // SEQ

_mesh = plsc.VectorSubcoreMesh(core_axis_name="c", subcore_axis_name="s")


@functools.partial(
    pl.kernel,
    mesh=_mesh,
    out_type=jax.ShapeDtypeStruct((TOTAL, EMBED), jnp.float32),
    scratch_types=[
        pltpu.VMEM((CHUNK,), jnp.int32),
        pltpu.VMEM((CHUNK, EMBED), jnp.float32),
        pltpu.VMEM((SEQ, EMBED), jnp.float32),
        pltpu.SemaphoreType.DMA,
    ],
    compiler_params=pltpu.CompilerParams(use_tc_tiling_on_sc=False),
)
def _embed_lookup(x_hbm, tok_hbm, pos_hbm, out_hbm, idx_v, rows_v, pos_v, sem):
    wid = lax.axis_index("s") * NC + lax.axis_index("c")
    base = wid * PER_W
    pltpu.sync_copy(pos_hbm, pos_v)

    def chunk_body(g, carry):
        cbase = base + g * CHUNK
        pltpu.sync_copy(x_hbm.at[pl.ds(cbase, CHUNK)], idx_v)
        pltpu.async_copy(tok_hbm.at[idx_v], rows_v, sem).wait()

        def pos_body(n, c2):
            p0 = pos_v[n, pl.ds(0, 16)]
            p1 = pos_v[n, pl.ds(16, 16)]

            def seq_body(s, c3):
                r = s * SEQ + n
                rows_v[r, pl.ds(0, 16)] += p0
                rows_v[r, pl.ds(16, 16)] += p1
                return c3

            return lax.fori_loop(0, SEQ_PER_CHUNK, seq_body, c2)

        lax.fori_loop(0, SEQ, pos_body, 0)
        pltpu.sync_copy(rows_v, out_hbm.at[pl.ds(cbase, CHUNK)])
        return carry

    lax.fori_loop(0, NCHUNK, chunk_body, 0)


def kernel(x, tok_table, pos_table):
    x_flat = x.reshape(TOTAL).astype(jnp.int32)
    out = _embed_lookup(x_flat, tok_table, pos_table)
    return out.reshape(BATCH, SEQ, EMBED)


# speedup vs baseline: 1.4715x; 1.0557x over previous
import functools

import jax
import jax.numpy as jnp
from jax import lax
from jax.experimental import pallas as pl
from jax.experimental.pallas import tpu as pltpu
from jax.experimental.pallas import tpu_sc as plsc

EMBED = 32
SEQ = 200
BATCH = 4096
TOTAL = BATCH * SEQ
NC = 2
NS = 16
NW = NC * NS
PER_W = TOTAL // NW
CHUNK = 800
NCHUNK = PER_W // CHUNK
SEQ_PER_CHUNK = CHUNK // SEQ

_mesh = plsc.VectorSubcoreMesh(core_axis_name="c", subcore_axis_name="s")


@functools.partial(
    pl.kernel,
    mesh=_mesh,
    out_type=jax.ShapeDtypeStruct((TOTAL, EMBED), jnp.float32),
    scratch_types=[
        pltpu.VMEM((CHUNK,), jnp.int32),
        pltpu.VMEM((CHUNK,), jnp.int32),
        pltpu.VMEM((CHUNK, EMBED), jnp.float32),
        pltpu.VMEM((CHUNK, EMBED), jnp.float32),
        pltpu.VMEM((SEQ, EMBED), jnp.float32),
        pltpu.SemaphoreType.DMA,
        pltpu.SemaphoreType.DMA,
        pltpu.SemaphoreType.DMA,
        pltpu.SemaphoreType.DMA,
        pltpu.SemaphoreType.DMA,
        pltpu.SemaphoreType.DMA,
    ],
    compiler_params=pltpu.CompilerParams(use_tc_tiling_on_sc=False),
)
def _embed_lookup(x_hbm, tok_hbm, pos_hbm, out_hbm,
                  idx0, idx1, rows0, rows1, pos_v,
                  s_i0, s_i1, s_g0, s_g1, s_o0, s_o1):
    idx = (idx0, idx1)
    rows = (rows0, rows1)
    s_i = (s_i0, s_i1)
    s_g = (s_g0, s_g1)
    s_o = (s_o0, s_o1)

    wid = lax.axis_index("s") * NC + lax.axis_index("c")
    base = wid * PER_W
    pltpu.sync_copy(pos_hbm, pos_v)

    def start_idx(g, b):
        pltpu.async_copy(x_hbm.at[pl.ds(base + g * CHUNK, CHUNK)], idx[b], s_i[b])

    def wait_idx(b):
        pltpu.make_async_copy(x_hbm.at[pl.ds(base, CHUNK)], idx[b], s_i[b]).wait()

    def start_gather(b):
        pltpu.async_copy(tok_hbm.at[idx[b]], rows[b], s_g[b])

    def wait_gather(b):
        pltpu.make_async_copy(tok_hbm.at[pl.ds(0, CHUNK)], rows[b], s_g[b]).wait()

    def start_out(g, b):
        pltpu.async_copy(rows[b], out_hbm.at[pl.ds(base + g * CHUNK, CHUNK)], s_o[b])

    def wait_out(b):
        pltpu.make_async_copy(rows[b], out_hbm.at[pl.ds(base, CHUNK)], s_o[b]).wait()

    def add_pos(b):
        rb = rows[b]

        def pos_body(n, c):
            p0 = pos_v[n, pl.ds(0, 16)]
            p1 = pos_v[n, pl.ds(16, 16)]
            for s in range(SEQ_PER_CHUNK):
                r = s * SEQ + n
                rb[r, pl.ds(0, 16)] += p0
                rb[r, pl.ds(16, 16)] += p1
            return c

        lax.fori_loop(0, SEQ, pos_body, 0, unroll=2)

    start_idx(0, 0)

    def outer(t, carry):
        for b in range(2):
            g = 2 * t + b
            o = 1 - b
            wait_idx(b)

            @pl.when(t > 0)
            def _():
                wait_out(b)

            start_gather(b)
            if b == 0:
                @pl.when(t > 0)
                def _():
                    wait_gather(o)

                start_idx(g + 1, o)

                @pl.when(t > 0)
                def _():
                    add_pos(o)
                    start_out(g - 1, o)
            else:
                wait_gather(o)

                @pl.when(t < NCHUNK // 2 - 1)
                def _():
                    start_idx(g + 1, o)

                add_pos(o)
                start_out(g - 1, o)
        return carry

    lax.fori_loop(0, NCHUNK // 2, outer, 0)

    wait_gather(1)
    add_pos(1)
    start_out(NCHUNK - 1, 1)
    wait_out(0)
    wait_out(1)


def kernel(x, tok_table, pos_table):
    x_flat = x.reshape(TOTAL).astype(jnp.int32)
    out = _embed_lookup(x_flat, tok_table, pos_table)
    return out.reshape(BATCH, SEQ, EMBED)


# speedup vs baseline: 1.4731x; 1.0011x over previous
import functools

import jax
import jax.numpy as jnp
from jax import lax
from jax.experimental import pallas as pl
from jax.experimental.pallas import tpu as pltpu
from jax.experimental.pallas import tpu_sc as plsc

EMBED = 32
SEQ = 200
BATCH = 4096
NC = 2
NS = 16
NW = NC * NS
ROWS_W = BATCH // NW
BS = 4
NCHUNK = ROWS_W // BS

_mesh = plsc.VectorSubcoreMesh(core_axis_name="c", subcore_axis_name="s")


@functools.partial(
    pl.kernel,
    mesh=_mesh,
    out_type=jax.ShapeDtypeStruct((BATCH, SEQ, EMBED), jnp.float32),
    scratch_types=[
        pltpu.VMEM((BS, SEQ), jnp.int32),
        pltpu.VMEM((BS, SEQ), jnp.int32),
        pltpu.VMEM((BS, SEQ, EMBED), jnp.float32),
        pltpu.VMEM((BS, SEQ, EMBED), jnp.float32),
        pltpu.VMEM((SEQ, EMBED), jnp.float32),
        pltpu.SemaphoreType.DMA,
        pltpu.SemaphoreType.DMA,
        pltpu.SemaphoreType.DMA,
        pltpu.SemaphoreType.DMA,
        pltpu.SemaphoreType.DMA,
        pltpu.SemaphoreType.DMA,
    ],
    compiler_params=pltpu.CompilerParams(use_tc_tiling_on_sc=False),
)
def _embed_lookup(x_hbm, tok_hbm, pos_hbm, out_hbm,
                  idx0, idx1, rows0, rows1, pos_v,
                  s_i0, s_i1, s_g0, s_g1, s_o0, s_o1):
    idx = (idx0, idx1)
    rows = (rows0, rows1)
    s_i = (s_i0, s_i1)
    s_g = (s_g0, s_g1)
    s_o = (s_o0, s_o1)

    wid = lax.axis_index("s") * NC + lax.axis_index("c")
    base = wid * ROWS_W
    pltpu.sync_copy(pos_hbm, pos_v)

    def start_idx(g, b):
        pltpu.async_copy(x_hbm.at[pl.ds(base + g * BS, BS)], idx[b], s_i[b])

    def wait_idx(b):
        pltpu.make_async_copy(x_hbm.at[pl.ds(0, BS)], idx[b], s_i[b]).wait()

    def start_gather(b):
        for s in range(BS):
            pltpu.async_copy(tok_hbm.at[idx[b].at[s]], rows[b].at[s], s_g[b])

    def wait_gather(b):
        for s in range(BS):
            pltpu.make_async_copy(tok_hbm.at[idx[b].at[s]], rows[b].at[s],
                                  s_g[b]).wait()

    def start_out(g, b):
        pltpu.async_copy(rows[b], out_hbm.at[pl.ds(base + g * BS, BS)], s_o[b])

    def wait_out(b):
        pltpu.make_async_copy(rows[b], out_hbm.at[pl.ds(0, BS)], s_o[b]).wait()

    def add_pos(b):
        rb = rows[b]

        def pos_body(n, c):
            p0 = pos_v[n, pl.ds(0, 16)]
            p1 = pos_v[n, pl.ds(16, 16)]
            for s in range(BS):
                rb[s, n, pl.ds(0, 16)] += p0
                rb[s, n, pl.ds(16, 16)] += p1
            return c

        lax.fori_loop(0, SEQ, pos_body, 0, unroll=2)

    start_idx(0, 0)

    def outer(t, carry):
        for b in range(2):
            g = 2 * t + b
            o = 1 - b
            wait_idx(b)

            @pl.when(t > 0)
            def _():
                wait_out(b)

            start_gather(b)
            if b == 0:
                @pl.when(t > 0)
                def _():
                    wait_gather(o)

                start_idx(g + 1, o)

                @pl.when(t > 0)
                def _():
                    add_pos(o)
                    start_out(g - 1, o)
            else:
                wait_gather(o)

                @pl.when(t < NCHUNK // 2 - 1)
                def _():
                    start_idx(g + 1, o)

                add_pos(o)
                start_out(g - 1, o)
        return carry

    lax.fori_loop(0, NCHUNK // 2, outer, 0)

    wait_gather(1)
    add_pos(1)
    start_out(NCHUNK - 1, 1)
    wait_out(0)
    wait_out(1)


def kernel(x, tok_table, pos_table):
    return _embed_lookup(x.astype(jnp.int32), tok_table, pos_table)
